# baseline (speedup 1.0000x reference)
import functools

import jax
import jax.numpy as jnp
from jax import lax
from jax.experimental import pallas as pl
from jax.experimental.pallas import tpu as pltpu
from jax.experimental.pallas import tpu_sc as plsc

DEPTH, NH, NP_, NL = 2, 8, 9, 3
NN_LIST = [16, 64]
B, N, C = 2, 1024, 256
DSH = 32
INNER = NH * DSH
S = 21504
HWS = ((128, 128), (64, 64), (32, 32))
LSTART = (0, 16384, 20480)
DFF = 512
NPTS = NH * NL * NP_
GRP = NL * NP_
NTAP = GRP * 4
NTAPP = 112

_F32_INF_BITS = 0x7F800000


def _cpe_body(cp6_ref, wpos_ref, o_ref):
    cp = cp6_ref[0]
    w = wpos_ref[...]
    o_ref[0] = cp[:, 0:1] * w[0:1, :] + cp[:, 1:2] * w[1:2, :]


def _cpe(cp6, wpos):
    return pl.pallas_call(
        _cpe_body,
        grid=(B,),
        in_specs=[
            pl.BlockSpec((1, N, 6), lambda b: (b, 0, 0)),
            pl.BlockSpec((2, C), lambda b: (0, 0)),
        ],
        out_specs=pl.BlockSpec((1, N, C), lambda b: (b, 0, 0)),
        out_shape=jax.ShapeDtypeStruct((B, N, C), jnp.float32),
    )(cp6, wpos)


def _ln_rows(x, s, b):
    m = jnp.mean(x, axis=-1, keepdims=True)
    v = jnp.mean((x - m) * (x - m), axis=-1, keepdims=True)
    return (x - m) / jnp.sqrt(v + 1e-5) * s + b


def _attn_body(k_sel, x_ref, cpe_ref, p3_ref, p3t_ref, lns_ref, lnb_ref,
               wq_ref, wkv_ref, wo_ref, bo_ref, o_ref, oacc_ref):
    x = x_ref[0]
    h = _ln_rows(x + cpe_ref[0], lns_ref[...], lnb_ref[...])

    p3 = p3_ref[0]
    p3t = p3t_ref[0]
    dx = p3[:, 0:1] - p3t[0:1, :]
    dy = p3[:, 1:2] - p3t[1:2, :]
    dz = p3[:, 2:3] - p3t[2:3, :]
    d2 = dx * dx + dy * dy + dz * dz

    bits = lax.bitcast_convert_type(d2, jnp.int32)

    def vstep(_, c):
        lo, hi = c
        mid = lo + lax.shift_right_logical(hi - lo, 1)
        cnt = jnp.sum((bits <= mid).astype(jnp.int32), axis=1, keepdims=True)
        take = cnt >= k_sel
        return jnp.where(take, lo, mid + 1), jnp.where(take, mid, hi)

    lo0 = jnp.zeros((N, 1), jnp.int32)
    hi0 = jnp.full((N, 1), _F32_INF_BITS, jnp.int32)
    _, t = lax.fori_loop(0, 31, vstep, (lo0, hi0))

    c_lt = jnp.sum((bits < t).astype(jnp.int32), axis=1, keepdims=True)
    need = k_sel - c_lt
    jj = lax.broadcasted_iota(jnp.int32, (N, N), 1)
    is_tie = bits == t

    def istep(_, c):
        ilo, ihi = c
        imid = ilo + lax.shift_right_logical(ihi - ilo, 1)
        cnt = jnp.sum((is_tie & (jj <= imid)).astype(jnp.int32), axis=1,
                      keepdims=True)
        take = cnt >= need
        return jnp.where(take, ilo, imid + 1), jnp.where(take, imid, ihi)

    ilo0 = jnp.zeros((N, 1), jnp.int32)
    ihi0 = jnp.full((N, 1), N - 1, jnp.int32)
    _, tie_hi = lax.fori_loop(0, 10, istep, (ilo0, ihi0))

    mask = (bits < t) | (is_tie & (jj <= tie_hi))

    q = jnp.dot(h, wq_ref[...], preferred_element_type=jnp.float32)
    kv = jnp.dot(h, wkv_ref[...], preferred_element_type=jnp.float32)
    kk = kv[:, :INNER]
    vv = kv[:, INNER:]

    scale = DSH ** -0.5
    for hd in range(NH):
        qh = q[:, hd * DSH:(hd + 1) * DSH]
        khd = kk[:, hd * DSH:(hd + 1) * DSH]
        vhd = vv[:, hd * DSH:(hd + 1) * DSH]
        dots = lax.dot_general(qh, khd, (((1,), (1,)), ((), ())),
                               preferred_element_type=jnp.float32) * scale
        dots = jnp.where(mask, dots, -1e30)
        mx = jnp.max(dots, axis=1, keepdims=True)
        e = jnp.where(mask, jnp.exp(dots - mx), 0.0)
        ssum = jnp.sum(e, axis=1, keepdims=True)
        attn = e / ssum
        oacc_ref[:, hd * DSH:(hd + 1) * DSH] = jnp.dot(
            attn, vhd, preferred_element_type=jnp.float32)

    o_ref[0] = (jnp.dot(oacc_ref[...], wo_ref[...],
                        preferred_element_type=jnp.float32)
                + bo_ref[...] + x)


def _attn_block(k_sel, x, cpe, p3, p3t, lns, lnb, wq, wkv, wo, bo):
    return pl.pallas_call(
        functools.partial(_attn_body, k_sel),
        grid=(B,),
        in_specs=[
            pl.BlockSpec((1, N, C), lambda b: (b, 0, 0)),
            pl.BlockSpec((1, N, C), lambda b: (b, 0, 0)),
            pl.BlockSpec((1, N, 3), lambda b: (b, 0, 0)),
            pl.BlockSpec((1, 3, N), lambda b: (b, 0, 0)),
            pl.BlockSpec((1, C), lambda b: (0, 0)),
            pl.BlockSpec((1, C), lambda b: (0, 0)),
            pl.BlockSpec((C, INNER), lambda b: (0, 0)),
            pl.BlockSpec((C, 2 * INNER), lambda b: (0, 0)),
            pl.BlockSpec((INNER, C), lambda b: (0, 0)),
            pl.BlockSpec((1, C), lambda b: (0, 0)),
        ],
        out_specs=pl.BlockSpec((1, N, C), lambda b: (b, 0, 0)),
        out_shape=jax.ShapeDtypeStruct((B, N, C), jnp.float32),
        scratch_shapes=[pltpu.VMEM((N, INNER), jnp.float32)],
    )(x, cpe, p3, p3t, lns, lnb, wq, wkv, wo, bo)


_SBLK = 1344


def _value_body(src_ref, lns_ref, lnb_ref, wv_ref, bv_ref, o_ref):
    sn = _ln_rows(src_ref[0], lns_ref[...], lnb_ref[...])
    o_ref[0] = jnp.dot(sn, wv_ref[...],
                       preferred_element_type=jnp.float32) + bv_ref[...]


def _value_proj(src, lns, lnb, wv, bv):
    return pl.pallas_call(
        _value_body,
        grid=(B, S // _SBLK),
        in_specs=[
            pl.BlockSpec((1, _SBLK, C), lambda b, s: (b, s, 0)),
            pl.BlockSpec((1, C), lambda b, s: (0, 0)),
            pl.BlockSpec((1, C), lambda b, s: (0, 0)),
            pl.BlockSpec((C, INNER), lambda b, s: (0, 0)),
            pl.BlockSpec((1, INNER), lambda b, s: (0, 0)),
        ],
        out_specs=pl.BlockSpec((1, _SBLK, INNER), lambda b, s: (b, s, 0)),
        out_shape=jax.ShapeDtypeStruct((B, S, INNER), jnp.float32),
    )(src, lns, lnb, wv, bv)


def _prep_body(x_ref, cpe_ref, cp6_ref, lns_ref, lnb_ref, wox_ref, woy_ref,
               box_ref, boy_ref, waw_ref, baw_ref, idx_ref, wts_ref):
    b = pl.program_id(0)
    qpc = _ln_rows(x_ref[0], lns_ref[...], lnb_ref[...]) + cpe_ref[0]
    cp6 = cp6_ref[0]

    offx = jnp.dot(qpc, wox_ref[...],
                   preferred_element_type=jnp.float32) + box_ref[...]
    offy = jnp.dot(qpc, woy_ref[...],
                   preferred_element_type=jnp.float32) + boy_ref[...]
    awl = jnp.dot(qpc, waw_ref[...],
                  preferred_element_type=jnp.float32) + baw_ref[...]

    a3 = awl.reshape(N, NH, GRP)
    a3 = a3 - jnp.max(a3, axis=-1, keepdims=True)
    e3 = jnp.exp(a3)
    aw = (e3 / jnp.sum(e3, axis=-1, keepdims=True)).reshape(N, NPTS)

    col = lax.broadcasted_iota(jnp.int32, (N, NPTS), 1)
    lvl = (col // NP_) % NL
    w_l = jnp.where(lvl == 0, float(HWS[0][1]),
                    jnp.where(lvl == 1, float(HWS[1][1]), float(HWS[2][1])))
    h_l = jnp.where(lvl == 0, float(HWS[0][0]),
                    jnp.where(lvl == 1, float(HWS[1][0]), float(HWS[2][0])))
    start = jnp.where(lvl == 0, LSTART[0],
                      jnp.where(lvl == 1, LSTART[1], LSTART[2]))
    refx = jnp.where(lvl == 0, cp6[:, 0:1],
                     jnp.where(lvl == 1, cp6[:, 2:3], cp6[:, 4:5]))
    refy = jnp.where(lvl == 0, cp6[:, 1:2],
                     jnp.where(lvl == 1, cp6[:, 3:4], cp6[:, 5:6]))

    gx = (refx + offx / w_l) * w_l - 0.5
    gy = (refy + offy / h_l) * h_l - 0.5
    x0 = jnp.floor(gx)
    y0 = jnp.floor(gy)
    wx1 = gx - x0
    wx0 = x0 + 1.0 - gx
    wy1 = gy - y0
    wy0 = y0 + 1.0 - gy

    wi = jnp.int32(w_l)
    hi_ = jnp.int32(h_l)
    hd = col // GRP

    def corner(xi, yi, wgt):
        valid = ((xi >= 0) & (xi <= w_l - 1.0)
                 & (yi >= 0) & (yi <= h_l - 1.0)).astype(jnp.float32)
        xc = jnp.clip(xi, 0.0, w_l - 1.0).astype(jnp.int32)
        yc = jnp.clip(yi, 0.0, h_l - 1.0).astype(jnp.int32)
        lin = yc * wi + xc
        rowid = (b * S + start + lin) * NH + hd
        return rowid, aw * wgt * valid

    i00, w00 = corner(x0, y0, wx0 * wy0)
    i10, w10 = corner(x0 + 1.0, y0, wx1 * wy0)
    i01, w01 = corner(x0, y0 + 1.0, wx0 * wy1)
    i11, w11 = corner(x0 + 1.0, y0 + 1.0, wx1 * wy1)

    zi = jnp.zeros((N, NTAPP - NTAP), jnp.int32)
    zw = jnp.zeros((N, NTAPP - NTAP), jnp.float32)
    for h in range(NH):
        sl = slice(h * GRP, (h + 1) * GRP)
        idx_ref[0, h] = jnp.concatenate(
            [i00[:, sl], i10[:, sl], i01[:, sl], i11[:, sl], zi], axis=1)
        wts_ref[0, h] = jnp.concatenate(
            [w00[:, sl], w10[:, sl], w01[:, sl], w11[:, sl], zw], axis=1)


def _prep(x, cpe, cp6, lns, lnb, wox, woy, box, boy, waw, baw):
    return pl.pallas_call(
        _prep_body,
        grid=(B,),
        in_specs=[
            pl.BlockSpec((1, N, C), lambda b: (b, 0, 0)),
            pl.BlockSpec((1, N, C), lambda b: (b, 0, 0)),
            pl.BlockSpec((1, N, 6), lambda b: (b, 0, 0)),
            pl.BlockSpec((1, C), lambda b: (0, 0)),
            pl.BlockSpec((1, C), lambda b: (0, 0)),
            pl.BlockSpec((C, NPTS), lambda b: (0, 0)),
            pl.BlockSpec((C, NPTS), lambda b: (0, 0)),
            pl.BlockSpec((1, NPTS), lambda b: (0, 0)),
            pl.BlockSpec((1, NPTS), lambda b: (0, 0)),
            pl.BlockSpec((C, NPTS), lambda b: (0, 0)),
            pl.BlockSpec((1, NPTS), lambda b: (0, 0)),
        ],
        out_specs=[
            pl.BlockSpec((1, NH, N, NTAPP), lambda b: (b, 0, 0, 0)),
            pl.BlockSpec((1, NH, N, NTAPP), lambda b: (b, 0, 0, 0)),
        ],
        out_shape=[
            jax.ShapeDtypeStruct((B, NH, N, NTAPP), jnp.int32),
            jax.ShapeDtypeStruct((B, NH, N, NTAPP), jnp.float32),
        ],
    )(x, cpe, cp6, lns, lnb, wox, woy, box, boy, waw, baw)


_ROWS = B * NH * N
_CH = 8


def _sc_gather(idx_flat, wts_flat, table):
    info = plsc.get_sparse_core_info()
    nw = info.num_cores * info.num_subcores
    rpw = _ROWS // nw
    mesh = plsc.VectorSubcoreMesh(core_axis_name="c", subcore_axis_name="s")

    nch = rpw // _CH

    @functools.partial(
        pl.kernel,
        out_type=jax.ShapeDtypeStruct((_ROWS, DSH), jnp.float32),
        mesh=mesh,
        compiler_params=pltpu.CompilerParams(
            needs_layout_passes=False, use_tc_tiling_on_sc=False),
        scratch_types=[
            pltpu.VMEM((2, _CH, NTAPP), jnp.int32),
            pltpu.VMEM((2, _CH, NTAPP), jnp.float32),
            pltpu.VMEM((2, _CH, NTAPP, DSH), jnp.float32),
            pltpu.VMEM((rpw, DSH), jnp.float32),
            pltpu.SemaphoreType.DMA,
            pltpu.SemaphoreType.DMA,
            pltpu.SemaphoreType.DMA,
        ],
    )
    def k(idx_hbm, wts_hbm, tab_hbm, out_hbm, idx_v, wts_v, rows_v, out_v,
          sem_m, sem_g0, sem_g1):
        wid = lax.axis_index("s") * info.num_cores + lax.axis_index("c")
        base_row = wid * rpw

        def stage_and_fire(c, buf, sem):
            base = base_row + c * _CH
            cpy1 = pltpu.async_copy(idx_hbm.at[pl.ds(base, _CH)],
                                    idx_v.at[buf], sem_m)
            cpy2 = pltpu.async_copy(wts_hbm.at[pl.ds(base, _CH)],
                                    wts_v.at[buf], sem_m)
            cpy1.wait()
            cpy2.wait()

            def fire(r, _):
                pltpu.async_copy(tab_hbm.at[idx_v.at[buf, r]],
                                 rows_v.at[buf, r], sem)
                return 0

            lax.fori_loop(0, _CH, fire, 0)

        def drain_and_compute(c, buf, sem):
            def drain(r, _):
                pltpu.make_async_copy(tab_hbm.at[idx_v.at[buf, r]],
                                     rows_v.at[buf, r], sem).wait()
                return 0

            lax.fori_loop(0, _CH, drain, 0)

            def row(r, _):
                a0 = [jnp.zeros((16,), jnp.float32) for _ in range(4)]
                a1 = [jnp.zeros((16,), jnp.float32) for _ in range(4)]
                rfull = jnp.full((16,), r, jnp.int32)
                bfull = jnp.full((16,), buf, jnp.int32)
                for j in range(NTAP):
                    w = plsc.load_gather(
                        wts_v, [bfull, rfull, jnp.full((16,), j, jnp.int32)])
                    t = j % 4
                    a0[t] = a0[t] + w * rows_v[buf, r, j, 0:16]
                    a1[t] = a1[t] + w * rows_v[buf, r, j, 16:32]
                out_v[c * _CH + r, 0:16] = (a0[0] + a0[1]) + (a0[2] + a0[3])
                out_v[c * _CH + r, 16:32] = (a1[0] + a1[1]) + (a1[2] + a1[3])
                return 0

            lax.fori_loop(0, _CH, row, 0)

        stage_and_fire(0, 0, sem_g0)

        def chunk(c, _):
            buf = lax.rem(c, 2)
            nbuf = 1 - buf

            @pl.when(c + 1 < nch)
            def _():
                lax.cond(nbuf == 0,
                         lambda: stage_and_fire(c + 1, 0, sem_g0),
                         lambda: stage_and_fire(c + 1, 1, sem_g1))

            lax.cond(buf == 0,
                     lambda: drain_and_compute(c, 0, sem_g0),
                     lambda: drain_and_compute(c, 1, sem_g1))
            return 0

        lax.fori_loop(0, nch, chunk, 0)
        pltpu.sync_copy(out_v, out_hbm.at[pl.ds(base_row, rpw)])

    return k(idx_flat, wts_flat, table)


def _tail_body(samp_ref, x_ref, wout_ref, bout_ref, lns_ref, lnb_ref,
               w1_ref, b1_ref, w2_ref, b2_ref, o_ref):
    samp = jnp.concatenate([samp_ref[0, h] for h in range(NH)], axis=1)
    x2 = (jnp.dot(samp, wout_ref[...], preferred_element_type=jnp.float32)
          + bout_ref[...] + x_ref[0])
    h2 = _ln_rows(x2, lns_ref[...], lnb_ref[...])
    ff = jax.nn.gelu(jnp.dot(h2, w1_ref[...],
                             preferred_element_type=jnp.float32) + b1_ref[...])
    o_ref[0] = (jnp.dot(ff, w2_ref[...], preferred_element_type=jnp.float32)
                + b2_ref[...] + x2)


def _tail(samp, x1, wout, bout, lns, lnb, w1, b1, w2, b2):
    return pl.pallas_call(
        _tail_body,
        grid=(B,),
        in_specs=[
            pl.BlockSpec((1, NH, N, DSH), lambda b: (b, 0, 0, 0)),
            pl.BlockSpec((1, N, C), lambda b: (b, 0, 0)),
            pl.BlockSpec((INNER, C), lambda b: (0, 0)),
            pl.BlockSpec((1, C), lambda b: (0, 0)),
            pl.BlockSpec((1, C), lambda b: (0, 0)),
            pl.BlockSpec((1, C), lambda b: (0, 0)),
            pl.BlockSpec((C, DFF), lambda b: (0, 0)),
            pl.BlockSpec((1, DFF), lambda b: (0, 0)),
            pl.BlockSpec((DFF, C), lambda b: (0, 0)),
            pl.BlockSpec((1, C), lambda b: (0, 0)),
        ],
        out_specs=pl.BlockSpec((1, N, C), lambda b: (b, 0, 0)),
        out_shape=jax.ShapeDtypeStruct((B, N, C), jnp.float32),
    )(samp, x1, wout, bout, lns, lnb, w1, b1, w2, b2)


def kernel(x, src, src_spatial_shapes, level_start_index, center_pos,
           center_pos3d, W_pos, ln1_s, ln1_b, Wq, Wkv, Wo, bo, ln2_s, ln2_b,
           W_off, b_off, W_aw, b_aw, W_v, b_v, W_out, b_out, ln3_s, ln3_b,
           W1, b1, W2, b2):
    cp6 = center_pos.reshape(B, N, NL * 2)
    p3 = center_pos3d
    p3t = jnp.swapaxes(center_pos3d, 1, 2)
    cpe = _cpe(cp6, W_pos)

    for i in range(DEPTH):
        x = _attn_block(NN_LIST[i], x, cpe, p3, p3t,
                        ln1_s[i].reshape(1, C), ln1_b[i].reshape(1, C),
                        Wq[i], Wkv[i], Wo[i], bo[i].reshape(1, C))

        value = _value_proj(src, ln2_s[i].reshape(1, C), ln2_b[i].reshape(1, C),
                            W_v[i], b_v[i].reshape(1, INNER))
        table = value.reshape(B * S * NH, DSH)

        wof = W_off[i].reshape(C, NH, NL, NP_, 2)
        wox = wof[..., 0].reshape(C, NPTS)
        woy = wof[..., 1].reshape(C, NPTS)
        bof = b_off[i].reshape(NH, NL, NP_, 2)
        box = bof[..., 0].reshape(1, NPTS)
        boy = bof[..., 1].reshape(1, NPTS)

        idx, wts = _prep(x, cpe, cp6, ln2_s[i].reshape(1, C),
                         ln2_b[i].reshape(1, C), wox, woy, box, boy,
                         W_aw[i], b_aw[i].reshape(1, NPTS))

        samp = _sc_gather(idx.reshape(_ROWS, NTAPP), wts.reshape(_ROWS, NTAPP),
                          table)

        x = _tail(samp.reshape(B, NH, N, DSH), x, W_out[i],
                  b_out[i].reshape(1, C), ln3_s[i].reshape(1, C),
                  ln3_b[i].reshape(1, C), W1[i], b1[i].reshape(1, DFF),
                  W2[i], b2[i].reshape(1, C))
    return x


# speedup vs baseline: 1.0049x; 1.0049x over previous
import functools

import jax
import jax.numpy as jnp
from jax import lax
from jax.experimental import pallas as pl
from jax.experimental.pallas import tpu as pltpu
from jax.experimental.pallas import tpu_sc as plsc

DEPTH, NH, NP_, NL = 2, 8, 9, 3
NN_LIST = [16, 64]
B, N, C = 2, 1024, 256
DSH = 32
INNER = NH * DSH
S = 21504
HWS = ((128, 128), (64, 64), (32, 32))
LSTART = (0, 16384, 20480)
DFF = 512
NPTS = NH * NL * NP_
GRP = NL * NP_
NTAP = GRP * 4
NTAPP = 112

_F32_INF_BITS = 0x7F800000


def _cpe_body(cp6_ref, wpos_ref, o_ref):
    cp = cp6_ref[0]
    w = wpos_ref[...]
    o_ref[0] = cp[:, 0:1] * w[0:1, :] + cp[:, 1:2] * w[1:2, :]


def _cpe(cp6, wpos):
    return pl.pallas_call(
        _cpe_body,
        grid=(B,),
        in_specs=[
            pl.BlockSpec((1, N, 6), lambda b: (b, 0, 0)),
            pl.BlockSpec((2, C), lambda b: (0, 0)),
        ],
        out_specs=pl.BlockSpec((1, N, C), lambda b: (b, 0, 0)),
        out_shape=jax.ShapeDtypeStruct((B, N, C), jnp.float32),
    )(cp6, wpos)


def _ln_rows(x, s, b):
    m = jnp.mean(x, axis=-1, keepdims=True)
    v = jnp.mean((x - m) * (x - m), axis=-1, keepdims=True)
    return (x - m) / jnp.sqrt(v + 1e-5) * s + b


def _attn_body(k_sel, x_ref, cpe_ref, p3_ref, p3t_ref, lns_ref, lnb_ref,
               wq_ref, wkv_ref, wo_ref, bo_ref, o_ref, oacc_ref):
    x = x_ref[0]
    h = _ln_rows(x + cpe_ref[0], lns_ref[...], lnb_ref[...])

    p3 = p3_ref[0]
    p3t = p3t_ref[0]
    dx = p3[:, 0:1] - p3t[0:1, :]
    dy = p3[:, 1:2] - p3t[1:2, :]
    dz = p3[:, 2:3] - p3t[2:3, :]
    d2 = dx * dx + dy * dy + dz * dz

    bits = lax.bitcast_convert_type(d2, jnp.int32)

    def vstep(_, c):
        lo, hi = c
        mid = lo + lax.shift_right_logical(hi - lo, 1)
        cnt = jnp.sum((bits <= mid).astype(jnp.int32), axis=1, keepdims=True)
        take = cnt >= k_sel
        return jnp.where(take, lo, mid + 1), jnp.where(take, mid, hi)

    lo0 = jnp.zeros((N, 1), jnp.int32)
    hi0 = jnp.full((N, 1), _F32_INF_BITS, jnp.int32)
    _, t = lax.fori_loop(0, 31, vstep, (lo0, hi0))

    c_lt = jnp.sum((bits < t).astype(jnp.int32), axis=1, keepdims=True)
    need = k_sel - c_lt
    jj = lax.broadcasted_iota(jnp.int32, (N, N), 1)
    is_tie = bits == t

    def istep(_, c):
        ilo, ihi = c
        imid = ilo + lax.shift_right_logical(ihi - ilo, 1)
        cnt = jnp.sum((is_tie & (jj <= imid)).astype(jnp.int32), axis=1,
                      keepdims=True)
        take = cnt >= need
        return jnp.where(take, ilo, imid + 1), jnp.where(take, imid, ihi)

    ilo0 = jnp.zeros((N, 1), jnp.int32)
    ihi0 = jnp.full((N, 1), N - 1, jnp.int32)
    _, tie_hi = lax.fori_loop(0, 10, istep, (ilo0, ihi0))

    mask = (bits < t) | (is_tie & (jj <= tie_hi))

    q = jnp.dot(h, wq_ref[...], preferred_element_type=jnp.float32)
    kv = jnp.dot(h, wkv_ref[...], preferred_element_type=jnp.float32)
    kk = kv[:, :INNER]
    vv = kv[:, INNER:]

    scale = DSH ** -0.5
    for hd in range(NH):
        qh = q[:, hd * DSH:(hd + 1) * DSH]
        khd = kk[:, hd * DSH:(hd + 1) * DSH]
        vhd = vv[:, hd * DSH:(hd + 1) * DSH]
        dots = lax.dot_general(qh, khd, (((1,), (1,)), ((), ())),
                               preferred_element_type=jnp.float32) * scale
        dots = jnp.where(mask, dots, -1e30)
        mx = jnp.max(dots, axis=1, keepdims=True)
        e = jnp.where(mask, jnp.exp(dots - mx), 0.0)
        ssum = jnp.sum(e, axis=1, keepdims=True)
        attn = e / ssum
        oacc_ref[:, hd * DSH:(hd + 1) * DSH] = jnp.dot(
            attn, vhd, preferred_element_type=jnp.float32)

    o_ref[0] = (jnp.dot(oacc_ref[...], wo_ref[...],
                        preferred_element_type=jnp.float32)
                + bo_ref[...] + x)


def _attn_block(k_sel, x, cpe, p3, p3t, lns, lnb, wq, wkv, wo, bo):
    return pl.pallas_call(
        functools.partial(_attn_body, k_sel),
        grid=(B,),
        in_specs=[
            pl.BlockSpec((1, N, C), lambda b: (b, 0, 0)),
            pl.BlockSpec((1, N, C), lambda b: (b, 0, 0)),
            pl.BlockSpec((1, N, 3), lambda b: (b, 0, 0)),
            pl.BlockSpec((1, 3, N), lambda b: (b, 0, 0)),
            pl.BlockSpec((1, C), lambda b: (0, 0)),
            pl.BlockSpec((1, C), lambda b: (0, 0)),
            pl.BlockSpec((C, INNER), lambda b: (0, 0)),
            pl.BlockSpec((C, 2 * INNER), lambda b: (0, 0)),
            pl.BlockSpec((INNER, C), lambda b: (0, 0)),
            pl.BlockSpec((1, C), lambda b: (0, 0)),
        ],
        out_specs=pl.BlockSpec((1, N, C), lambda b: (b, 0, 0)),
        out_shape=jax.ShapeDtypeStruct((B, N, C), jnp.float32),
        scratch_shapes=[pltpu.VMEM((N, INNER), jnp.float32)],
    )(x, cpe, p3, p3t, lns, lnb, wq, wkv, wo, bo)


_SBLK = 1344


def _value_body(src_ref, lns_ref, lnb_ref, wv_ref, bv_ref, o_ref):
    sn = _ln_rows(src_ref[0], lns_ref[...], lnb_ref[...])
    o_ref[0] = jnp.dot(sn, wv_ref[...],
                       preferred_element_type=jnp.float32) + bv_ref[...]


def _value_proj(src, lns, lnb, wv, bv):
    return pl.pallas_call(
        _value_body,
        grid=(B, S // _SBLK),
        in_specs=[
            pl.BlockSpec((1, _SBLK, C), lambda b, s: (b, s, 0)),
            pl.BlockSpec((1, C), lambda b, s: (0, 0)),
            pl.BlockSpec((1, C), lambda b, s: (0, 0)),
            pl.BlockSpec((C, INNER), lambda b, s: (0, 0)),
            pl.BlockSpec((1, INNER), lambda b, s: (0, 0)),
        ],
        out_specs=pl.BlockSpec((1, _SBLK, INNER), lambda b, s: (b, s, 0)),
        out_shape=jax.ShapeDtypeStruct((B, S, INNER), jnp.float32),
    )(src, lns, lnb, wv, bv)


def _prep_body(x_ref, cpe_ref, cp6_ref, lns_ref, lnb_ref, wox_ref, woy_ref,
               box_ref, boy_ref, waw_ref, baw_ref, idx_ref, wts_ref):
    b = pl.program_id(0)
    qpc = _ln_rows(x_ref[0], lns_ref[...], lnb_ref[...]) + cpe_ref[0]
    cp6 = cp6_ref[0]

    offx = jnp.dot(qpc, wox_ref[...],
                   preferred_element_type=jnp.float32) + box_ref[...]
    offy = jnp.dot(qpc, woy_ref[...],
                   preferred_element_type=jnp.float32) + boy_ref[...]
    awl = jnp.dot(qpc, waw_ref[...],
                  preferred_element_type=jnp.float32) + baw_ref[...]

    a3 = awl.reshape(N, NH, GRP)
    a3 = a3 - jnp.max(a3, axis=-1, keepdims=True)
    e3 = jnp.exp(a3)
    aw = (e3 / jnp.sum(e3, axis=-1, keepdims=True)).reshape(N, NPTS)

    col = lax.broadcasted_iota(jnp.int32, (N, NPTS), 1)
    lvl = (col // NP_) % NL
    w_l = jnp.where(lvl == 0, float(HWS[0][1]),
                    jnp.where(lvl == 1, float(HWS[1][1]), float(HWS[2][1])))
    h_l = jnp.where(lvl == 0, float(HWS[0][0]),
                    jnp.where(lvl == 1, float(HWS[1][0]), float(HWS[2][0])))
    start = jnp.where(lvl == 0, LSTART[0],
                      jnp.where(lvl == 1, LSTART[1], LSTART[2]))
    refx = jnp.where(lvl == 0, cp6[:, 0:1],
                     jnp.where(lvl == 1, cp6[:, 2:3], cp6[:, 4:5]))
    refy = jnp.where(lvl == 0, cp6[:, 1:2],
                     jnp.where(lvl == 1, cp6[:, 3:4], cp6[:, 5:6]))

    gx = (refx + offx / w_l) * w_l - 0.5
    gy = (refy + offy / h_l) * h_l - 0.5
    x0 = jnp.floor(gx)
    y0 = jnp.floor(gy)
    wx1 = gx - x0
    wx0 = x0 + 1.0 - gx
    wy1 = gy - y0
    wy0 = y0 + 1.0 - gy

    wi = jnp.int32(w_l)
    hi_ = jnp.int32(h_l)
    hd = col // GRP

    def corner(xi, yi, wgt):
        valid = ((xi >= 0) & (xi <= w_l - 1.0)
                 & (yi >= 0) & (yi <= h_l - 1.0)).astype(jnp.float32)
        xc = jnp.clip(xi, 0.0, w_l - 1.0).astype(jnp.int32)
        yc = jnp.clip(yi, 0.0, h_l - 1.0).astype(jnp.int32)
        lin = yc * wi + xc
        rowid = (b * S + start + lin) * NH + hd
        return rowid, aw * wgt * valid

    i00, w00 = corner(x0, y0, wx0 * wy0)
    i10, w10 = corner(x0 + 1.0, y0, wx1 * wy0)
    i01, w01 = corner(x0, y0 + 1.0, wx0 * wy1)
    i11, w11 = corner(x0 + 1.0, y0 + 1.0, wx1 * wy1)

    zi = jnp.zeros((N, NTAPP - NTAP), jnp.int32)
    zw = jnp.zeros((N, NTAPP - NTAP), jnp.float32)
    for h in range(NH):
        sl = slice(h * GRP, (h + 1) * GRP)
        idx_ref[0, h] = jnp.concatenate(
            [i00[:, sl], i10[:, sl], i01[:, sl], i11[:, sl], zi], axis=1)
        wts_ref[0, h] = jnp.concatenate(
            [w00[:, sl], w10[:, sl], w01[:, sl], w11[:, sl], zw], axis=1)


def _prep(x, cpe, cp6, lns, lnb, wox, woy, box, boy, waw, baw):
    return pl.pallas_call(
        _prep_body,
        grid=(B,),
        in_specs=[
            pl.BlockSpec((1, N, C), lambda b: (b, 0, 0)),
            pl.BlockSpec((1, N, C), lambda b: (b, 0, 0)),
            pl.BlockSpec((1, N, 6), lambda b: (b, 0, 0)),
            pl.BlockSpec((1, C), lambda b: (0, 0)),
            pl.BlockSpec((1, C), lambda b: (0, 0)),
            pl.BlockSpec((C, NPTS), lambda b: (0, 0)),
            pl.BlockSpec((C, NPTS), lambda b: (0, 0)),
            pl.BlockSpec((1, NPTS), lambda b: (0, 0)),
            pl.BlockSpec((1, NPTS), lambda b: (0, 0)),
            pl.BlockSpec((C, NPTS), lambda b: (0, 0)),
            pl.BlockSpec((1, NPTS), lambda b: (0, 0)),
        ],
        out_specs=[
            pl.BlockSpec((1, NH, N, NTAPP), lambda b: (b, 0, 0, 0)),
            pl.BlockSpec((1, NH, N, NTAPP), lambda b: (b, 0, 0, 0)),
        ],
        out_shape=[
            jax.ShapeDtypeStruct((B, NH, N, NTAPP), jnp.int32),
            jax.ShapeDtypeStruct((B, NH, N, NTAPP), jnp.float32),
        ],
    )(x, cpe, cp6, lns, lnb, wox, woy, box, boy, waw, baw)


_ROWS = B * NH * N
_CH = 8


def _sc_gather(idx_flat, wts_flat, table):
    info = plsc.get_sparse_core_info()
    nw = info.num_cores * info.num_subcores
    rpw = _ROWS // nw
    mesh = plsc.VectorSubcoreMesh(core_axis_name="c", subcore_axis_name="s")

    nch = rpw // _CH

    @functools.partial(
        pl.kernel,
        out_type=jax.ShapeDtypeStruct((_ROWS, DSH), jnp.float32),
        mesh=mesh,
        compiler_params=pltpu.CompilerParams(
            needs_layout_passes=False, use_tc_tiling_on_sc=False),
        scratch_types=[
            pltpu.VMEM((2, _CH, NTAPP), jnp.int32),
            pltpu.VMEM((2, _CH, NTAPP), jnp.float32),
            pltpu.VMEM((2, _CH, NTAPP, DSH), jnp.float32),
            pltpu.VMEM((rpw, DSH), jnp.float32),
            pltpu.SemaphoreType.DMA,
            pltpu.SemaphoreType.DMA,
            pltpu.SemaphoreType.DMA,
        ],
    )
    def k(idx_hbm, wts_hbm, tab_hbm, out_hbm, idx_v, wts_v, rows_v, out_v,
          sem_m, sem_g0, sem_g1):
        wid = lax.axis_index("s") * info.num_cores + lax.axis_index("c")
        base_row = wid * rpw

        def stage_and_fire(c, buf, sem):
            base = base_row + c * _CH
            cpy1 = pltpu.async_copy(idx_hbm.at[pl.ds(base, _CH)],
                                    idx_v.at[buf], sem_m)
            cpy2 = pltpu.async_copy(wts_hbm.at[pl.ds(base, _CH)],
                                    wts_v.at[buf], sem_m)
            cpy1.wait()
            cpy2.wait()

            def fire(r, _):
                pltpu.async_copy(tab_hbm.at[idx_v.at[buf, r]],
                                 rows_v.at[buf, r], sem)
                return 0

            lax.fori_loop(0, _CH, fire, 0)

        def drain_and_compute(c, buf, sem):
            def drain(r, _):
                pltpu.make_async_copy(tab_hbm.at[idx_v.at[buf, r]],
                                     rows_v.at[buf, r], sem).wait()
                return 0

            lax.fori_loop(0, _CH, drain, 0)

            def row(r, _):
                out_v[c * _CH + r, 0:16] = rows_v[buf, r, 0, 0:16]
                out_v[c * _CH + r, 16:32] = rows_v[buf, r, 0, 16:32]
                return 0

            lax.fori_loop(0, _CH, row, 0)

        stage_and_fire(0, 0, sem_g0)

        def chunk(c, _):
            buf = lax.rem(c, 2)
            nbuf = 1 - buf

            @pl.when(c + 1 < nch)
            def _():
                lax.cond(nbuf == 0,
                         lambda: stage_and_fire(c + 1, 0, sem_g0),
                         lambda: stage_and_fire(c + 1, 1, sem_g1))

            lax.cond(buf == 0,
                     lambda: drain_and_compute(c, 0, sem_g0),
                     lambda: drain_and_compute(c, 1, sem_g1))
            return 0

        lax.fori_loop(0, nch, chunk, 0)
        pltpu.sync_copy(out_v, out_hbm.at[pl.ds(base_row, rpw)])

    return k(idx_flat, wts_flat, table)


def _tail_body(samp_ref, x_ref, wout_ref, bout_ref, lns_ref, lnb_ref,
               w1_ref, b1_ref, w2_ref, b2_ref, o_ref):
    samp = jnp.concatenate([samp_ref[0, h] for h in range(NH)], axis=1)
    x2 = (jnp.dot(samp, wout_ref[...], preferred_element_type=jnp.float32)
          + bout_ref[...] + x_ref[0])
    h2 = _ln_rows(x2, lns_ref[...], lnb_ref[...])
    ff = jax.nn.gelu(jnp.dot(h2, w1_ref[...],
                             preferred_element_type=jnp.float32) + b1_ref[...])
    o_ref[0] = (jnp.dot(ff, w2_ref[...], preferred_element_type=jnp.float32)
                + b2_ref[...] + x2)


def _tail(samp, x1, wout, bout, lns, lnb, w1, b1, w2, b2):
    return pl.pallas_call(
        _tail_body,
        grid=(B,),
        in_specs=[
            pl.BlockSpec((1, NH, N, DSH), lambda b: (b, 0, 0, 0)),
            pl.BlockSpec((1, N, C), lambda b: (b, 0, 0)),
            pl.BlockSpec((INNER, C), lambda b: (0, 0)),
            pl.BlockSpec((1, C), lambda b: (0, 0)),
            pl.BlockSpec((1, C), lambda b: (0, 0)),
            pl.BlockSpec((1, C), lambda b: (0, 0)),
            pl.BlockSpec((C, DFF), lambda b: (0, 0)),
            pl.BlockSpec((1, DFF), lambda b: (0, 0)),
            pl.BlockSpec((DFF, C), lambda b: (0, 0)),
            pl.BlockSpec((1, C), lambda b: (0, 0)),
        ],
        out_specs=pl.BlockSpec((1, N, C), lambda b: (b, 0, 0)),
        out_shape=jax.ShapeDtypeStruct((B, N, C), jnp.float32),
    )(samp, x1, wout, bout, lns, lnb, w1, b1, w2, b2)


def kernel(x, src, src_spatial_shapes, level_start_index, center_pos,
           center_pos3d, W_pos, ln1_s, ln1_b, Wq, Wkv, Wo, bo, ln2_s, ln2_b,
           W_off, b_off, W_aw, b_aw, W_v, b_v, W_out, b_out, ln3_s, ln3_b,
           W1, b1, W2, b2):
    cp6 = center_pos.reshape(B, N, NL * 2)
    p3 = center_pos3d
    p3t = jnp.swapaxes(center_pos3d, 1, 2)
    cpe = _cpe(cp6, W_pos)

    for i in range(DEPTH):
        x = _attn_block(NN_LIST[i], x, cpe, p3, p3t,
                        ln1_s[i].reshape(1, C), ln1_b[i].reshape(1, C),
                        Wq[i], Wkv[i], Wo[i], bo[i].reshape(1, C))

        value = _value_proj(src, ln2_s[i].reshape(1, C), ln2_b[i].reshape(1, C),
                            W_v[i], b_v[i].reshape(1, INNER))
        table = value.reshape(B * S * NH, DSH)

        wof = W_off[i].reshape(C, NH, NL, NP_, 2)
        wox = wof[..., 0].reshape(C, NPTS)
        woy = wof[..., 1].reshape(C, NPTS)
        bof = b_off[i].reshape(NH, NL, NP_, 2)
        box = bof[..., 0].reshape(1, NPTS)
        boy = bof[..., 1].reshape(1, NPTS)

        idx, wts = _prep(x, cpe, cp6, ln2_s[i].reshape(1, C),
                         ln2_b[i].reshape(1, C), wox, woy, box, boy,
                         W_aw[i], b_aw[i].reshape(1, NPTS))

        samp = _sc_gather(idx.reshape(_ROWS, NTAPP), wts.reshape(_ROWS, NTAPP),
                          table)

        x = _tail(samp.reshape(B, NH, N, DSH), x, W_out[i],
                  b_out[i].reshape(1, C), ln3_s[i].reshape(1, C),
                  ln3_b[i].reshape(1, C), W1[i], b1[i].reshape(1, DFF),
                  W2[i], b2[i].reshape(1, C))
    return x


# speedup vs baseline: 1.4591x; 1.4519x over previous
import functools

import jax
import jax.numpy as jnp
from jax import lax
from jax.experimental import pallas as pl
from jax.experimental.pallas import tpu as pltpu
from jax.experimental.pallas import tpu_sc as plsc

DEPTH, NH, NP_, NL = 2, 8, 9, 3
NN_LIST = [16, 64]
B, N, C = 2, 1024, 256
DSH = 32
INNER = NH * DSH
S = 21504
HWS = ((128, 128), (64, 64), (32, 32))
LSTART = (0, 16384, 20480)
DFF = 512
NPTS = NH * NL * NP_
GRP = NL * NP_
NTAP = GRP * 4
NTAPP = 112
NTAPI = 28
PD = 4 * DSH

_F32_INF_BITS = 0x7F800000


def _cpe_body(cp6_ref, wpos_ref, o_ref):
    cp = cp6_ref[0]
    w = wpos_ref[...]
    o_ref[0] = cp[:, 0:1] * w[0:1, :] + cp[:, 1:2] * w[1:2, :]


def _cpe(cp6, wpos):
    return pl.pallas_call(
        _cpe_body,
        grid=(B,),
        in_specs=[
            pl.BlockSpec((1, N, 6), lambda b: (b, 0, 0)),
            pl.BlockSpec((2, C), lambda b: (0, 0)),
        ],
        out_specs=pl.BlockSpec((1, N, C), lambda b: (b, 0, 0)),
        out_shape=jax.ShapeDtypeStruct((B, N, C), jnp.float32),
    )(cp6, wpos)


def _ln_rows(x, s, b):
    m = jnp.mean(x, axis=-1, keepdims=True)
    v = jnp.mean((x - m) * (x - m), axis=-1, keepdims=True)
    return (x - m) / jnp.sqrt(v + 1e-5) * s + b


def _attn_body(k_sel, x_ref, cpe_ref, p3_ref, p3t_ref, lns_ref, lnb_ref,
               wq_ref, wkv_ref, wo_ref, bo_ref, o_ref, oacc_ref):
    x = x_ref[0]
    h = _ln_rows(x + cpe_ref[0], lns_ref[...], lnb_ref[...])

    p3 = p3_ref[0]
    p3t = p3t_ref[0]
    dx = p3[:, 0:1] - p3t[0:1, :]
    dy = p3[:, 1:2] - p3t[1:2, :]
    dz = p3[:, 2:3] - p3t[2:3, :]
    d2 = dx * dx + dy * dy + dz * dz

    bits = lax.bitcast_convert_type(d2, jnp.int32)

    def vstep(_, c):
        lo, hi = c
        mid = lo + lax.shift_right_logical(hi - lo, 1)
        cnt = jnp.sum((bits <= mid).astype(jnp.int32), axis=1, keepdims=True)
        take = cnt >= k_sel
        return jnp.where(take, lo, mid + 1), jnp.where(take, mid, hi)

    lo0 = jnp.zeros((N, 1), jnp.int32)
    hi0 = jnp.full((N, 1), _F32_INF_BITS, jnp.int32)
    _, t = lax.fori_loop(0, 31, vstep, (lo0, hi0))

    c_lt = jnp.sum((bits < t).astype(jnp.int32), axis=1, keepdims=True)
    need = k_sel - c_lt
    jj = lax.broadcasted_iota(jnp.int32, (N, N), 1)
    is_tie = bits == t

    def istep(_, c):
        ilo, ihi = c
        imid = ilo + lax.shift_right_logical(ihi - ilo, 1)
        cnt = jnp.sum((is_tie & (jj <= imid)).astype(jnp.int32), axis=1,
                      keepdims=True)
        take = cnt >= need
        return jnp.where(take, ilo, imid + 1), jnp.where(take, imid, ihi)

    ilo0 = jnp.zeros((N, 1), jnp.int32)
    ihi0 = jnp.full((N, 1), N - 1, jnp.int32)
    _, tie_hi = lax.fori_loop(0, 10, istep, (ilo0, ihi0))

    mask = (bits < t) | (is_tie & (jj <= tie_hi))

    q = jnp.dot(h, wq_ref[...], preferred_element_type=jnp.float32)
    kv = jnp.dot(h, wkv_ref[...], preferred_element_type=jnp.float32)
    kk = kv[:, :INNER]
    vv = kv[:, INNER:]

    scale = DSH ** -0.5
    for hd in range(NH):
        qh = q[:, hd * DSH:(hd + 1) * DSH]
        khd = kk[:, hd * DSH:(hd + 1) * DSH]
        vhd = vv[:, hd * DSH:(hd + 1) * DSH]
        dots = lax.dot_general(qh, khd, (((1,), (1,)), ((), ())),
                               preferred_element_type=jnp.float32) * scale
        dots = jnp.where(mask, dots, -1e30)
        mx = jnp.max(dots, axis=1, keepdims=True)
        e = jnp.where(mask, jnp.exp(dots - mx), 0.0)
        ssum = jnp.sum(e, axis=1, keepdims=True)
        attn = e / ssum
        oacc_ref[:, hd * DSH:(hd + 1) * DSH] = jnp.dot(
            attn, vhd, preferred_element_type=jnp.float32)

    o_ref[0] = (jnp.dot(oacc_ref[...], wo_ref[...],
                        preferred_element_type=jnp.float32)
                + bo_ref[...] + x)


def _attn_block(k_sel, x, cpe, p3, p3t, lns, lnb, wq, wkv, wo, bo):
    return pl.pallas_call(
        functools.partial(_attn_body, k_sel),
        grid=(B,),
        in_specs=[
            pl.BlockSpec((1, N, C), lambda b: (b, 0, 0)),
            pl.BlockSpec((1, N, C), lambda b: (b, 0, 0)),
            pl.BlockSpec((1, N, 3), lambda b: (b, 0, 0)),
            pl.BlockSpec((1, 3, N), lambda b: (b, 0, 0)),
            pl.BlockSpec((1, C), lambda b: (0, 0)),
            pl.BlockSpec((1, C), lambda b: (0, 0)),
            pl.BlockSpec((C, INNER), lambda b: (0, 0)),
            pl.BlockSpec((C, 2 * INNER), lambda b: (0, 0)),
            pl.BlockSpec((INNER, C), lambda b: (0, 0)),
            pl.BlockSpec((1, C), lambda b: (0, 0)),
        ],
        out_specs=pl.BlockSpec((1, N, C), lambda b: (b, 0, 0)),
        out_shape=jax.ShapeDtypeStruct((B, N, C), jnp.float32),
        scratch_shapes=[pltpu.VMEM((N, INNER), jnp.float32)],
    )(x, cpe, p3, p3t, lns, lnb, wq, wkv, wo, bo)


_SBLK = 1344


def _value_body(src_ref, lns_ref, lnb_ref, wv_ref, bv_ref, o_ref):
    sn = _ln_rows(src_ref[0], lns_ref[...], lnb_ref[...])
    mm = jnp.dot(sn, wv_ref[...],
                 preferred_element_type=jnp.float32) + bv_ref[...]
    for h in range(NH):
        o_ref[0, h] = mm[:, h * DSH:(h + 1) * DSH]


def _value_proj(src, lns, lnb, wv, bv):
    return pl.pallas_call(
        _value_body,
        grid=(B, S // _SBLK),
        in_specs=[
            pl.BlockSpec((1, _SBLK, C), lambda b, s: (b, s, 0)),
            pl.BlockSpec((1, C), lambda b, s: (0, 0)),
            pl.BlockSpec((1, C), lambda b, s: (0, 0)),
            pl.BlockSpec((C, INNER), lambda b, s: (0, 0)),
            pl.BlockSpec((1, INNER), lambda b, s: (0, 0)),
        ],
        out_specs=pl.BlockSpec((1, NH, _SBLK, DSH), lambda b, s: (b, 0, s, 0)),
        out_shape=jax.ShapeDtypeStruct((B, NH, S, DSH), jnp.float32),
    )(src, lns, lnb, wv, bv)


def _patch_body(wl, v_ref, vn_ref, o_ref):
    v = v_ref[0, 0]
    vn = vn_ref[0, 0]
    s1 = jnp.concatenate([v[1:], vn[:1]], axis=0)
    sw = jnp.concatenate([v[wl:], vn[:wl]], axis=0)
    sw1 = jnp.concatenate([v[wl + 1:], vn[:wl + 1]], axis=0)
    o_ref[0, 0] = jnp.concatenate([v, s1, sw, sw1], axis=1)


def _patch_table(value2, lvl):
    (hl, wl), st = HWS[lvl], LSTART[lvl]
    n = hl * wl
    blk = min(n, 4096)
    nb = n // blk
    sb = st // blk

    return pl.pallas_call(
        functools.partial(_patch_body, wl),
        grid=(B, NH, nb),
        in_specs=[
            pl.BlockSpec((1, 1, blk, DSH), lambda b, h, k: (b, h, sb + k, 0)),
            pl.BlockSpec((1, 1, blk, DSH),
                         lambda b, h, k: (b, h, sb + jnp.minimum(k + 1, nb - 1),
                                          0)),
        ],
        out_specs=pl.BlockSpec((1, 1, blk, PD), lambda b, h, k: (b, h, k, 0)),
        out_shape=jax.ShapeDtypeStruct((B, NH, n, PD), jnp.float32),
    )(value2, value2)


def _prep_body(x_ref, cpe_ref, cp6_ref, lns_ref, lnb_ref, wox_ref, woy_ref,
               box_ref, boy_ref, waw_ref, baw_ref, idx0_ref, idx1_ref,
               idx2_ref, wts_ref):
    idx_refs = (idx0_ref, idx1_ref, idx2_ref)
    b = pl.program_id(0)
    qpc = _ln_rows(x_ref[0], lns_ref[...], lnb_ref[...]) + cpe_ref[0]
    cp6 = cp6_ref[0]

    offx = jnp.dot(qpc, wox_ref[...],
                   preferred_element_type=jnp.float32) + box_ref[...]
    offy = jnp.dot(qpc, woy_ref[...],
                   preferred_element_type=jnp.float32) + boy_ref[...]
    awl = jnp.dot(qpc, waw_ref[...],
                  preferred_element_type=jnp.float32) + baw_ref[...]

    a3 = awl.reshape(N, NH, GRP)
    a3 = a3 - jnp.max(a3, axis=-1, keepdims=True)
    e3 = jnp.exp(a3)
    aw = (e3 / jnp.sum(e3, axis=-1, keepdims=True)).reshape(N, NPTS)

    col = lax.broadcasted_iota(jnp.int32, (N, NPTS), 1)
    lvl = (col // NP_) % NL
    w_l = jnp.where(lvl == 0, float(HWS[0][1]),
                    jnp.where(lvl == 1, float(HWS[1][1]), float(HWS[2][1])))
    h_l = jnp.where(lvl == 0, float(HWS[0][0]),
                    jnp.where(lvl == 1, float(HWS[1][0]), float(HWS[2][0])))
    refx = jnp.where(lvl == 0, cp6[:, 0:1],
                     jnp.where(lvl == 1, cp6[:, 2:3], cp6[:, 4:5]))
    refy = jnp.where(lvl == 0, cp6[:, 1:2],
                     jnp.where(lvl == 1, cp6[:, 3:4], cp6[:, 5:6]))

    gx = (refx + offx / w_l) * w_l - 0.5
    gy = (refy + offy / h_l) * h_l - 0.5
    x0 = jnp.floor(gx)
    y0 = jnp.floor(gy)
    wx1 = gx - x0
    wx0 = x0 + 1.0 - gx
    wy1 = gy - y0
    wy0 = y0 + 1.0 - gy

    wi = jnp.int32(w_l)
    hd = col // GRP

    xb = jnp.clip(x0, 0.0, w_l - 2.0)
    yb = jnp.clip(y0, 0.0, h_l - 2.0)
    vx0 = ((x0 >= 0) & (x0 <= w_l - 1.0)).astype(jnp.float32)
    vx1 = ((x0 >= -1.0) & (x0 <= w_l - 2.0)).astype(jnp.float32)
    vy0 = ((y0 >= 0) & (y0 <= h_l - 1.0)).astype(jnp.float32)
    vy1 = ((y0 >= -1.0) & (y0 <= h_l - 2.0)).astype(jnp.float32)

    def eqf(u, v):
        return (u == v).astype(jnp.float32)

    a0 = wx0 * vx0 * eqf(xb, x0) + wx1 * vx1 * eqf(xb, x0 + 1.0)
    a1 = wx0 * vx0 * eqf(xb + 1.0, x0) + wx1 * vx1 * eqf(xb + 1.0, x0 + 1.0)
    b0 = wy0 * vy0 * eqf(yb, y0) + wy1 * vy1 * eqf(yb, y0 + 1.0)
    b1 = wy0 * vy0 * eqf(yb + 1.0, y0) + wy1 * vy1 * eqf(yb + 1.0, y0 + 1.0)

    hw_l = jnp.where(lvl == 0, HWS[0][0] * HWS[0][1],
                     jnp.where(lvl == 1, HWS[1][0] * HWS[1][1],
                               HWS[2][0] * HWS[2][1]))
    lin = yb.astype(jnp.int32) * wi + xb.astype(jnp.int32)
    rowid = (b * NH + hd) * hw_l + lin
    w00 = aw * b0 * a0
    w01 = aw * b0 * a1
    w10 = aw * b1 * a0
    w11 = aw * b1 * a1

    zw = jnp.zeros((N, 4), jnp.float32)
    for h in range(NH):
        sl = slice(h * GRP, (h + 1) * GRP)
        for lv in range(NL):
            s2 = slice(h * GRP + lv * NP_, h * GRP + (lv + 1) * NP_)
            idx_refs[lv][0, h] = rowid[:, s2]
        wts_ref[0, h] = jnp.concatenate(
            [w00[:, sl], w01[:, sl], w10[:, sl], w11[:, sl], zw], axis=1)


def _prep(x, cpe, cp6, lns, lnb, wox, woy, box, boy, waw, baw):
    return pl.pallas_call(
        _prep_body,
        grid=(B,),
        in_specs=[
            pl.BlockSpec((1, N, C), lambda b: (b, 0, 0)),
            pl.BlockSpec((1, N, C), lambda b: (b, 0, 0)),
            pl.BlockSpec((1, N, 6), lambda b: (b, 0, 0)),
            pl.BlockSpec((1, C), lambda b: (0, 0)),
            pl.BlockSpec((1, C), lambda b: (0, 0)),
            pl.BlockSpec((C, NPTS), lambda b: (0, 0)),
            pl.BlockSpec((C, NPTS), lambda b: (0, 0)),
            pl.BlockSpec((1, NPTS), lambda b: (0, 0)),
            pl.BlockSpec((1, NPTS), lambda b: (0, 0)),
            pl.BlockSpec((C, NPTS), lambda b: (0, 0)),
            pl.BlockSpec((1, NPTS), lambda b: (0, 0)),
        ],
        out_specs=[
            pl.BlockSpec((1, NH, N, NP_), lambda b: (b, 0, 0, 0)),
            pl.BlockSpec((1, NH, N, NP_), lambda b: (b, 0, 0, 0)),
            pl.BlockSpec((1, NH, N, NP_), lambda b: (b, 0, 0, 0)),
            pl.BlockSpec((1, NH, N, NTAPP), lambda b: (b, 0, 0, 0)),
        ],
        out_shape=[
            jax.ShapeDtypeStruct((B, NH, N, NP_), jnp.int32),
            jax.ShapeDtypeStruct((B, NH, N, NP_), jnp.int32),
            jax.ShapeDtypeStruct((B, NH, N, NP_), jnp.int32),
            jax.ShapeDtypeStruct((B, NH, N, NTAPP), jnp.float32),
        ],
    )(x, cpe, cp6, lns, lnb, wox, woy, box, boy, waw, baw)


_ROWS = B * NH * N
_CH = 8


def _sc_gather(idxs, wts_flat, tabs):
    info = plsc.get_sparse_core_info()
    nw = info.num_cores * info.num_subcores
    rpw = _ROWS // nw
    mesh = plsc.VectorSubcoreMesh(core_axis_name="c", subcore_axis_name="s")

    nch = rpw // _CH

    @functools.partial(
        pl.kernel,
        out_type=jax.ShapeDtypeStruct((_ROWS, DSH), jnp.float32),
        mesh=mesh,
        compiler_params=pltpu.CompilerParams(
            needs_layout_passes=False, use_tc_tiling_on_sc=False),
        scratch_types=[
            [pltpu.VMEM((2, _CH, NP_), jnp.int32) for _ in range(NL)],
            pltpu.VMEM((2, _CH, NTAPP), jnp.float32),
            [pltpu.VMEM((2, _CH, NP_, PD), jnp.float32) for _ in range(NL)],
            pltpu.VMEM((rpw, DSH), jnp.float32),
            pltpu.SemaphoreType.DMA,
            pltpu.SemaphoreType.DMA,
            pltpu.SemaphoreType.DMA,
        ],
    )
    def k(idx0_hbm, idx1_hbm, idx2_hbm, wts_hbm, t0_hbm, t1_hbm, t2_hbm,
          out_hbm, idx_vs, wts_v, rows_vs, out_v, sem_m, sem_g0, sem_g1):
        idx_hbms = (idx0_hbm, idx1_hbm, idx2_hbm)
        tab_hbms = (t0_hbm, t1_hbm, t2_hbm)
        wid = lax.axis_index("s") * info.num_cores + lax.axis_index("c")
        base_row = wid * rpw

        def stage_and_fire(c, buf, sem):
            base = base_row + c * _CH
            cpys = [pltpu.async_copy(idx_hbms[lv].at[pl.ds(base, _CH)],
                                     idx_vs[lv].at[buf], sem_m)
                    for lv in range(NL)]
            cpyw = pltpu.async_copy(wts_hbm.at[pl.ds(base, _CH)],
                                    wts_v.at[buf], sem_m)
            for cp in cpys:
                cp.wait()
            cpyw.wait()

            def fire(r, _):
                for lv in range(NL):
                    pltpu.async_copy(tab_hbms[lv].at[idx_vs[lv].at[buf, r]],
                                     rows_vs[lv].at[buf, r], sem)
                return 0

            lax.fori_loop(0, _CH, fire, 0)

        def drain_and_compute(c, buf, sem):
            def drain(r, _):
                for lv in range(NL):
                    pltpu.make_async_copy(
                        tab_hbms[lv].at[idx_vs[lv].at[buf, r]],
                        rows_vs[lv].at[buf, r], sem).wait()
                return 0

            lax.fori_loop(0, _CH, drain, 0)

            def row(r, _):
                a0 = [jnp.zeros((16,), jnp.float32) for _ in range(4)]
                a1 = [jnp.zeros((16,), jnp.float32) for _ in range(4)]
                rfull = jnp.full((16,), r, jnp.int32)
                bfull = jnp.full((16,), buf, jnp.int32)
                for lv in range(NL):
                    for p in range(NP_):
                        j = lv * NP_ + p
                        for sl in range(4):
                            w = plsc.load_gather(
                                wts_v,
                                [bfull, rfull,
                                 jnp.full((16,), sl * GRP + j, jnp.int32)])
                            lo = sl * DSH
                            a0[sl] = (a0[sl]
                                      + w * rows_vs[lv][buf, r, p, lo:lo + 16])
                            a1[sl] = (a1[sl]
                                      + w * rows_vs[lv][buf, r, p,
                                                        lo + 16:lo + 32])
                out_v[c * _CH + r, 0:16] = (a0[0] + a0[1]) + (a0[2] + a0[3])
                out_v[c * _CH + r, 16:32] = (a1[0] + a1[1]) + (a1[2] + a1[3])
                return 0

            lax.fori_loop(0, _CH, row, 0)

        stage_and_fire(0, 0, sem_g0)

        def chunk(c, _):
            buf = lax.rem(c, 2)
            nbuf = 1 - buf

            @pl.when(c + 1 < nch)
            def _():
                lax.cond(nbuf == 0,
                         lambda: stage_and_fire(c + 1, 0, sem_g0),
                         lambda: stage_and_fire(c + 1, 1, sem_g1))

            lax.cond(buf == 0,
                     lambda: drain_and_compute(c, 0, sem_g0),
                     lambda: drain_and_compute(c, 1, sem_g1))
            return 0

        lax.fori_loop(0, nch, chunk, 0)
        pltpu.sync_copy(out_v, out_hbm.at[pl.ds(base_row, rpw)])

    return k(idxs[0], idxs[1], idxs[2], wts_flat, tabs[0], tabs[1], tabs[2])


def _tail_body(samp_ref, x_ref, wout_ref, bout_ref, lns_ref, lnb_ref,
               w1_ref, b1_ref, w2_ref, b2_ref, o_ref):
    samp = jnp.concatenate([samp_ref[0, h] for h in range(NH)], axis=1)
    x2 = (jnp.dot(samp, wout_ref[...], preferred_element_type=jnp.float32)
          + bout_ref[...] + x_ref[0])
    h2 = _ln_rows(x2, lns_ref[...], lnb_ref[...])
    ff = jax.nn.gelu(jnp.dot(h2, w1_ref[...],
                             preferred_element_type=jnp.float32) + b1_ref[...])
    o_ref[0] = (jnp.dot(ff, w2_ref[...], preferred_element_type=jnp.float32)
                + b2_ref[...] + x2)


def _tail(samp, x1, wout, bout, lns, lnb, w1, b1, w2, b2):
    return pl.pallas_call(
        _tail_body,
        grid=(B,),
        in_specs=[
            pl.BlockSpec((1, NH, N, DSH), lambda b: (b, 0, 0, 0)),
            pl.BlockSpec((1, N, C), lambda b: (b, 0, 0)),
            pl.BlockSpec((INNER, C), lambda b: (0, 0)),
            pl.BlockSpec((1, C), lambda b: (0, 0)),
            pl.BlockSpec((1, C), lambda b: (0, 0)),
            pl.BlockSpec((1, C), lambda b: (0, 0)),
            pl.BlockSpec((C, DFF), lambda b: (0, 0)),
            pl.BlockSpec((1, DFF), lambda b: (0, 0)),
            pl.BlockSpec((DFF, C), lambda b: (0, 0)),
            pl.BlockSpec((1, C), lambda b: (0, 0)),
        ],
        out_specs=pl.BlockSpec((1, N, C), lambda b: (b, 0, 0)),
        out_shape=jax.ShapeDtypeStruct((B, N, C), jnp.float32),
    )(samp, x1, wout, bout, lns, lnb, w1, b1, w2, b2)


def kernel(x, src, src_spatial_shapes, level_start_index, center_pos,
           center_pos3d, W_pos, ln1_s, ln1_b, Wq, Wkv, Wo, bo, ln2_s, ln2_b,
           W_off, b_off, W_aw, b_aw, W_v, b_v, W_out, b_out, ln3_s, ln3_b,
           W1, b1, W2, b2):
    cp6 = center_pos.reshape(B, N, NL * 2)
    p3 = center_pos3d
    p3t = jnp.swapaxes(center_pos3d, 1, 2)
    cpe = _cpe(cp6, W_pos)

    for i in range(DEPTH):
        x = _attn_block(NN_LIST[i], x, cpe, p3, p3t,
                        ln1_s[i].reshape(1, C), ln1_b[i].reshape(1, C),
                        Wq[i], Wkv[i], Wo[i], bo[i].reshape(1, C))

        value2 = _value_proj(src, ln2_s[i].reshape(1, C),
                             ln2_b[i].reshape(1, C), W_v[i],
                             b_v[i].reshape(1, INNER))
        tabs = [_patch_table(value2, lv).reshape(B * NH * HWS[lv][0]
                                                 * HWS[lv][1], PD)
                for lv in range(NL)]

        wof = W_off[i].reshape(C, NH, NL, NP_, 2)
        wox = wof[..., 0].reshape(C, NPTS)
        woy = wof[..., 1].reshape(C, NPTS)
        bof = b_off[i].reshape(NH, NL, NP_, 2)
        box = bof[..., 0].reshape(1, NPTS)
        boy = bof[..., 1].reshape(1, NPTS)

        idx0, idx1, idx2, wts = _prep(x, cpe, cp6, ln2_s[i].reshape(1, C),
                         ln2_b[i].reshape(1, C), wox, woy, box, boy,
                         W_aw[i], b_aw[i].reshape(1, NPTS))

        samp = _sc_gather([ix.reshape(_ROWS, NP_) for ix in (idx0, idx1, idx2)],
                          wts.reshape(_ROWS, NTAPP), tabs)

        x = _tail(samp.reshape(B, NH, N, DSH), x, W_out[i],
                  b_out[i].reshape(1, C), ln3_s[i].reshape(1, C),
                  ln3_b[i].reshape(1, C), W1[i], b1[i].reshape(1, DFF),
                  W2[i], b2[i].reshape(1, C))
    return x
